# baseline (speedup 1.0000x reference)
import functools

import jax
import jax.numpy as jnp
from jax import lax
from jax.experimental import pallas as pl
from jax.experimental.pallas import tpu as pltpu
from jax.experimental.pallas import tpu_sc as plsc

B, P, C = 32, 8732, 21
G = 8
PP = 8960
CP = 24
P4 = P * 4
PP4 = PP * 4
NCHUNK = PP // 16


def _dense_body(conf_ref, lab_ref, pred_ref, gt_ref, negkey_ref,
                stats_ref):
    x = conf_ref[...]
    lab = lab_ref[...]
    m = jnp.max(x, axis=1, keepdims=True)
    e = jnp.exp(x - m)
    s = jnp.sum(e, axis=1, keepdims=True)
    lse = m + jnp.log(s)
    bg = lse - x[:, 0:1, :]
    iota_c = lax.broadcasted_iota(jnp.int32, (G, CP, PP), 1)
    conf_lab = jnp.sum(jnp.where(iota_c == lab, x, 0.0), axis=1, keepdims=True)
    ce = lse - conf_lab
    posm = lab > 0
    negm = lab == 0
    nk = jnp.where(negm, bg, jnp.float32(-jnp.inf))
    negkey_ref[...] = nk[:, 0, :]
    pos_ce = jnp.sum(jnp.where(posm, ce, 0.0), axis=(1, 2), keepdims=True)
    npos = jnp.sum(jnp.where(posm, 1.0, 0.0), axis=(1, 2), keepdims=True)
    nneg = jnp.sum(jnp.where(negm, 1.0, 0.0), axis=(1, 2), keepdims=True)
    d = pred_ref[...] - gt_ref[...]
    ad = jnp.abs(d)
    sl1 = jnp.where(ad < 1.0, 0.5 * d * d, ad - 0.5)
    sl1s = jnp.sum(jnp.where(posm, sl1, 0.0), axis=(1, 2), keepdims=True)
    row = lax.broadcasted_iota(jnp.int32, (G, 8, 128), 1)
    stats_ref[...] = jnp.where(
        row == 0, pos_ce,
        jnp.where(row == 1, npos,
                  jnp.where(row == 2, nneg,
                            jnp.where(row == 3, sl1s, 0.0))))


def _dense_call(conf_t, lab3, pred_t, gt_t, interpret=False):
    return pl.pallas_call(
        _dense_body,
        grid=(B // G,),
        in_specs=[
            pl.BlockSpec((G, CP, PP), lambda b: (b, 0, 0)),
            pl.BlockSpec((G, 1, PP), lambda b: (b, 0, 0)),
            pl.BlockSpec((G, 8, PP), lambda b: (b, 0, 0)),
            pl.BlockSpec((G, 8, PP), lambda b: (b, 0, 0)),
        ],
        out_specs=[
            pl.BlockSpec((G, PP), lambda b: (b, 0)),
            pl.BlockSpec((G, 8, 128), lambda b: (b, 0, 0)),
        ],
        out_shape=[
            jax.ShapeDtypeStruct((B, PP), jnp.float32),
            jax.ShapeDtypeStruct((B, 8, 128), jnp.float32),
        ],
        compiler_params=pltpu.CompilerParams(
            allow_input_fusion=[True, True, True, True]),
        interpret=interpret,
    )(conf_t, lab3, pred_t, gt_t)


_UNROLL = 8


def _count_ge(key_v, cand):
    def cnt_body(i, acc):
        base = i * (16 * _UNROLL)
        for j in range(_UNROLL):
            m = key_v[pl.ds(base + j * 16, 16)] >= cand
            acc = acc + plsc.all_reduce_population_count(m)
        return acc
    return lax.fori_loop(0, NCHUNK // _UNROLL, cnt_body,
                         jnp.zeros((16,), jnp.int32))


def _mining_body(val_hbm, stats_hbm, out_hbm, key_v, val_v, np_v,
                 nn_v, out_v):
    info = plsc.get_sparse_core_info()
    w = lax.axis_index("s") * info.num_cores + lax.axis_index("c")
    pltpu.sync_copy(val_hbm.at[w], val_v)
    pltpu.sync_copy(stats_hbm.at[w, 1, pl.ds(0, 16)], np_v)
    pltpu.sync_copy(stats_hbm.at[w, 2, pl.ds(0, 16)], nn_v)
    kvec = jnp.minimum(3.0 * np_v[...], nn_v[...]).astype(jnp.int32)

    def mk_body(i, carry):
        base = i * (16 * _UNROLL)
        for j in range(_UNROLL):
            bits = plsc.bitcast(val_v[pl.ds(base + j * 16, 16)], jnp.int32)
            key_v[pl.ds(base + j * 16, 16)] = jnp.where(
                bits < 0, bits ^ jnp.int32(0x7FFFFFFF), bits)
        return carry
    lax.fori_loop(0, NCHUNK // _UNROLL, mk_body, 0)

    cnt0 = _count_ge(key_v, jnp.zeros((16,), jnp.int32))
    ans = jnp.where(cnt0 >= kvec, jnp.int32(0), jnp.int32(-2147483648))
    for b in range(30, -1, -1):
        cand = ans | jnp.int32(1 << b)
        cntv = _count_ge(key_v, cand)
        ans = jnp.where(cntv >= kvec, cand, ans)

    def fin_body(i, carry):
        sacc, eacc, cgt, ceq = carry
        base = i * (16 * _UNROLL)
        for j in range(_UNROLL):
            key = key_v[pl.ds(base + j * 16, 16)]
            x = val_v[pl.ds(base + j * 16, 16)]
            mgt = key > ans
            meq = key == ans
            sacc = sacc + jnp.where(mgt, x, 0.0)
            eacc = eacc + jnp.where(meq, x, 0.0)
            cgt = cgt + plsc.all_reduce_population_count(mgt)
            ceq = ceq + plsc.all_reduce_population_count(meq)
        return (sacc, eacc, cgt, ceq)
    sacc, eacc, cgt, ceq = lax.fori_loop(
        0, NCHUNK // _UNROLL, fin_body,
        (jnp.zeros((16,), jnp.float32), jnp.zeros((16,), jnp.float32),
         jnp.zeros((16,), jnp.int32), jnp.zeros((16,), jnp.int32)))

    frac = (kvec - cgt).astype(jnp.float32) / ceq.astype(jnp.float32)
    out_v[...] = jnp.where(kvec > 0, sacc + eacc * frac, 0.0)
    pltpu.sync_copy(out_v, out_hbm.at[w])


def _mining_call(negkey2d, stats):
    mesh = plsc.VectorSubcoreMesh(core_axis_name="c", subcore_axis_name="s")
    fn = functools.partial(
        pl.kernel,
        out_type=jax.ShapeDtypeStruct((B, 16), jnp.float32),
        mesh=mesh,
        compiler_params=pltpu.CompilerParams(needs_layout_passes=False),
        scratch_types=[
            pltpu.VMEM((PP,), jnp.int32),
            pltpu.VMEM((PP,), jnp.float32),
            pltpu.VMEM((16,), jnp.float32),
            pltpu.VMEM((16,), jnp.float32),
            pltpu.VMEM((16,), jnp.float32),
        ],
    )(_mining_body)
    return fn(negkey2d, stats)


def kernel(confidence, predicted_locations, labels, gt_locations):
    conf_p = jnp.pad(confidence.astype(jnp.float32),
                     ((0, 0), (0, PP - P), (0, CP - C)),
                     constant_values=-1e30)
    conf_t = jnp.transpose(conf_p, (0, 2, 1))
    lab_p = jnp.pad(labels.astype(jnp.int32), ((0, 0), (0, PP - P)),
                    constant_values=-1)
    lab3 = lab_p.reshape(B, 1, PP)
    pred_t = jnp.transpose(
        jnp.pad(predicted_locations.astype(jnp.float32),
                ((0, 0), (0, PP - P), (0, 4))), (0, 2, 1))
    gt_t = jnp.transpose(
        jnp.pad(gt_locations.astype(jnp.float32),
                ((0, 0), (0, PP - P), (0, 4))), (0, 2, 1))

    negkey, stats = _dense_call(conf_t, lab3, pred_t, gt_t)
    pos_ce = stats[:, 0, 0]
    npos = stats[:, 1, 0]
    sl1 = stats[:, 3, 0]

    neg_sum = jnp.sum(_mining_call(negkey, stats), axis=1)

    num_pos = jnp.sum(npos)
    classification_loss = (jnp.sum(pos_ce) + jnp.sum(neg_sum)) / num_pos
    smooth_l1_loss = jnp.sum(sl1) / num_pos
    return (smooth_l1_loss, classification_loss)


# speedup vs baseline: 1.2227x; 1.2227x over previous
import functools

import jax
import jax.numpy as jnp
from jax import lax
from jax.experimental import pallas as pl
from jax.experimental.pallas import tpu as pltpu
from jax.experimental.pallas import tpu_sc as plsc

B, P, C = 32, 8732, 21
G = 4
PP = 8960
CP = 24
P4 = P * 4
PP4 = PP * 4
NCHUNK = PP // 16


def _dense_body(conf_ref, lab_ref, pred_ref, gt_ref, negkey_ref,
                stats_ref):
    x = conf_ref[...]
    lab = lab_ref[...]
    m = jnp.max(x, axis=1, keepdims=True)
    e = jnp.exp(x - m)
    s = jnp.sum(e, axis=1, keepdims=True)
    lse = m + jnp.log(s)
    bg = lse - x[:, 0:1, :]
    iota_c = lax.broadcasted_iota(jnp.int32, (G, CP, PP), 1)
    conf_lab = jnp.sum(jnp.where(iota_c == lab, x, 0.0), axis=1, keepdims=True)
    ce = lse - conf_lab
    posm = lab > 0
    negm = lab == 0
    negkey_ref[...] = jnp.where(negm, bg, jnp.float32(-jnp.inf))
    pos_ce = jnp.sum(jnp.where(posm, ce, 0.0), axis=(1, 2), keepdims=True)
    npos = jnp.sum(jnp.where(posm, 1.0, 0.0), axis=(1, 2), keepdims=True)
    nneg = jnp.sum(jnp.where(negm, 1.0, 0.0), axis=(1, 2), keepdims=True)
    d = pred_ref[...] - gt_ref[...]
    ad = jnp.abs(d)
    sl1 = jnp.where(ad < 1.0, 0.5 * d * d, ad - 0.5)
    sl1s = jnp.sum(jnp.where(posm, sl1, 0.0), axis=(1, 2), keepdims=True)
    row = lax.broadcasted_iota(jnp.int32, (G, 8, 128), 1)
    stats_ref[...] = jnp.where(
        row == 0, pos_ce,
        jnp.where(row == 1, npos,
                  jnp.where(row == 2, nneg,
                            jnp.where(row == 3, sl1s, 0.0))))


def _dense_call(conf_t, lab3, pred_t, gt_t, interpret=False):
    return pl.pallas_call(
        _dense_body,
        grid=(B // G,),
        in_specs=[
            pl.BlockSpec((G, CP, PP), lambda b: (b, 0, 0)),
            pl.BlockSpec((G, 1, PP), lambda b: (b, 0, 0)),
            pl.BlockSpec((G, 8, PP), lambda b: (b, 0, 0)),
            pl.BlockSpec((G, 8, PP), lambda b: (b, 0, 0)),
        ],
        out_specs=[
            pl.BlockSpec((G, 1, PP), lambda b: (b, 0, 0)),
            pl.BlockSpec((G, 8, 128), lambda b: (b, 0, 0)),
        ],
        out_shape=[
            jax.ShapeDtypeStruct((B, 1, PP), jnp.float32),
            jax.ShapeDtypeStruct((B, 8, 128), jnp.float32),
        ],
        compiler_params=pltpu.CompilerParams(
            allow_input_fusion=[True, True, True, True]),
        interpret=interpret,
    )(conf_t, lab3, pred_t, gt_t)


_UNROLL = 8


def _count_ge(key_v, cand):
    def cnt_body(i, acc):
        base = i * (16 * _UNROLL)
        for j in range(_UNROLL):
            m = key_v[pl.ds(base + j * 16, 16)] >= cand
            acc = acc + plsc.all_reduce_population_count(m)
        return acc
    return lax.fori_loop(0, NCHUNK // _UNROLL, cnt_body,
                         jnp.zeros((16,), jnp.int32))


def _mining_body(val_hbm, stats_hbm, out_hbm, key_v, val_v, np_v,
                 nn_v, out_v):
    info = plsc.get_sparse_core_info()
    w = lax.axis_index("s") * info.num_cores + lax.axis_index("c")
    pltpu.sync_copy(val_hbm.at[w], val_v)
    pltpu.sync_copy(stats_hbm.at[w, 1, pl.ds(0, 16)], np_v)
    pltpu.sync_copy(stats_hbm.at[w, 2, pl.ds(0, 16)], nn_v)
    kvec = jnp.minimum(3.0 * np_v[...], nn_v[...]).astype(jnp.int32)

    def mk_body(i, carry):
        base = i * (16 * _UNROLL)
        for j in range(_UNROLL):
            bits = plsc.bitcast(val_v[pl.ds(base + j * 16, 16)], jnp.int32)
            key_v[pl.ds(base + j * 16, 16)] = jnp.where(
                bits < 0, bits ^ jnp.int32(0x7FFFFFFF), bits)
        return carry
    lax.fori_loop(0, NCHUNK // _UNROLL, mk_body, 0)

    cnt0 = _count_ge(key_v, jnp.zeros((16,), jnp.int32))
    ans = jnp.where(cnt0 >= kvec, jnp.int32(0), jnp.int32(-2147483648))
    for b in range(30, -1, -1):
        cand = ans | jnp.int32(1 << b)
        cntv = _count_ge(key_v, cand)
        ans = jnp.where(cntv >= kvec, cand, ans)

    def fin_body(i, carry):
        sacc, eacc, cgt, ceq = carry
        base = i * (16 * _UNROLL)
        for j in range(_UNROLL):
            key = key_v[pl.ds(base + j * 16, 16)]
            x = val_v[pl.ds(base + j * 16, 16)]
            mgt = key > ans
            meq = key == ans
            sacc = sacc + jnp.where(mgt, x, 0.0)
            eacc = eacc + jnp.where(meq, x, 0.0)
            cgt = cgt + plsc.all_reduce_population_count(mgt)
            ceq = ceq + plsc.all_reduce_population_count(meq)
        return (sacc, eacc, cgt, ceq)
    sacc, eacc, cgt, ceq = lax.fori_loop(
        0, NCHUNK // _UNROLL, fin_body,
        (jnp.zeros((16,), jnp.float32), jnp.zeros((16,), jnp.float32),
         jnp.zeros((16,), jnp.int32), jnp.zeros((16,), jnp.int32)))

    frac = (kvec - cgt).astype(jnp.float32) / ceq.astype(jnp.float32)
    out_v[...] = jnp.where(kvec > 0, sacc + eacc * frac, 0.0)
    pltpu.sync_copy(out_v, out_hbm.at[w])


def _mining_call(negkey2d, stats):
    mesh = plsc.VectorSubcoreMesh(core_axis_name="c", subcore_axis_name="s")
    fn = functools.partial(
        pl.kernel,
        out_type=jax.ShapeDtypeStruct((B, 16), jnp.float32),
        mesh=mesh,
        compiler_params=pltpu.CompilerParams(needs_layout_passes=False),
        scratch_types=[
            pltpu.VMEM((PP,), jnp.int32),
            pltpu.VMEM((PP,), jnp.float32),
            pltpu.VMEM((16,), jnp.float32),
            pltpu.VMEM((16,), jnp.float32),
            pltpu.VMEM((16,), jnp.float32),
        ],
    )(_mining_body)
    return fn(negkey2d, stats)


def kernel(confidence, predicted_locations, labels, gt_locations):
    conf_p = jnp.pad(confidence.astype(jnp.float32),
                     ((0, 0), (0, PP - P), (0, CP - C)),
                     constant_values=-1e30)
    conf_t = jnp.transpose(conf_p, (0, 2, 1))
    lab_p = jnp.pad(labels.astype(jnp.int32), ((0, 0), (0, PP - P)),
                    constant_values=-1)
    lab3 = lab_p.reshape(B, 1, PP)
    pred_t = jnp.transpose(
        jnp.pad(predicted_locations.astype(jnp.float32),
                ((0, 0), (0, PP - P), (0, 4))), (0, 2, 1))
    gt_t = jnp.transpose(
        jnp.pad(gt_locations.astype(jnp.float32),
                ((0, 0), (0, PP - P), (0, 4))), (0, 2, 1))

    negkey, stats = _dense_call(conf_t, lab3, pred_t, gt_t)
    pos_ce = stats[:, 0, 0]
    npos = stats[:, 1, 0]
    sl1 = stats[:, 3, 0]

    neg_sum = jnp.sum(_mining_call(negkey.reshape(B, PP), stats), axis=1)

    num_pos = jnp.sum(npos)
    classification_loss = (jnp.sum(pos_ce) + jnp.sum(neg_sum)) / num_pos
    smooth_l1_loss = jnp.sum(sl1) / num_pos
    return (smooth_l1_loss, classification_loss)
